# baseline (speedup 1.0000x reference)
import functools
import math

import jax
import jax.numpy as jnp
from jax import lax
from jax.experimental import pallas as pl
from jax.experimental.pallas import tpu as pltpu
from jax.experimental.pallas import tpu_sc as plsc

D = 768
SCALE = math.sqrt(D)
LANES = 16
NC, NS = 2, 16
NW = NC * NS
CHUNK = 64
HC = CHUNK // 2


def _emb_kernel(B):
    bpw = B // NW
    nchunk = bpw // CHUNK
    assert nchunk >= 4 and nchunk % 2 == 0
    mesh = plsc.VectorSubcoreMesh(core_axis_name="c", subcore_axis_name="s")

    @functools.partial(
        pl.kernel,
        mesh=mesh,
        out_type=jax.ShapeDtypeStruct((B, D), jnp.float32),
        scratch_types=[
            pltpu.VMEM((bpw,), jnp.int32),
            pltpu.VMEM((2, CHUNK, D), jnp.float32),
            pltpu.SemaphoreType.DMA,
            pltpu.SemaphoreType.DMA,
            pltpu.SemaphoreType.DMA,
            pltpu.SemaphoreType.DMA,
        ],
    )
    def k(idx_hbm, table_hbm, out_hbm, idx_v, rows, g0, g1, s0, s1):
        gsem = (g0, g1)
        ssem = (s0, s1)
        wid = lax.axis_index("s") * NC + lax.axis_index("c")
        base = wid * bpw
        pltpu.sync_copy(idx_hbm.at[pl.ds(base, bpw)], idx_v)

        def gather(i, b):
            return pltpu.make_async_copy(
                table_hbm.at[idx_v.at[pl.ds(i * CHUNK, CHUNK)]],
                rows.at[b], gsem[b],
            )

        def gather_half(i, b, h):
            return pltpu.make_async_copy(
                table_hbm.at[idx_v.at[pl.ds(i * CHUNK + h * HC, HC)]],
                rows.at[b, pl.ds(h * HC, HC)], gsem[b],
            )

        def scatter_half(i, b, h):
            return pltpu.make_async_copy(
                rows.at[b, pl.ds(h * HC, HC)],
                out_hbm.at[pl.ds(base + i * CHUNK + h * HC, HC)], ssem[b],
            )

        def scale_half(b, h):
            def row_body(r2, c):
                for u in range(2):
                    for j in range(D // LANES):
                        sl = pl.ds(j * LANES, LANES)
                        rows[b, 2 * r2 + u, sl] = rows[b, 2 * r2 + u, sl] * SCALE
                return c

            lax.fori_loop(h * HC // 2, (h + 1) * HC // 2, row_body, 0)

        def process(i, b):
            for h in (0, 1):
                scale_half(b, h)
                scatter_half(i, b, h).start()

        def wait_chunk_scatter(i, b):
            for h in (0, 1):
                scatter_half(i, b, h).wait()

        gather_half(0, 0, 0).start()
        gather_half(0, 0, 1).start()
        gather(1, 1).start()
        for h in (0, 1):
            gather_half(0, 0, h).wait()
            scale_half(0, h)
            scatter_half(0, 0, h).start()

        def pair_body(t, carry):
            for b in (1, 0):
                i = 2 * t + 1 + (1 - b)
                gather(i, b).wait()
                wait_chunk_scatter(i - 1, 1 - b)
                gather(i + 1, 1 - b).start()
                process(i, b)
            return carry

        lax.fori_loop(0, nchunk // 2 - 1, pair_body, 0)

        gather(nchunk - 1, 1).wait()
        wait_chunk_scatter(nchunk - 2, 0)
        process(nchunk - 1, 1)
        wait_chunk_scatter(nchunk - 1, 1)

    return k


def kernel(x, weight):
    b, s = x.shape
    idx = x.reshape(-1).astype(jnp.int32)
    out = _emb_kernel(b * s)(idx, weight)
    return out.reshape(b, s, D)


# speedup vs baseline: 1.0337x; 1.0337x over previous
import functools
import math

import jax
import jax.numpy as jnp
from jax import lax
from jax.experimental import pallas as pl
from jax.experimental.pallas import tpu as pltpu
from jax.experimental.pallas import tpu_sc as plsc

D = 768
SCALE = math.sqrt(D)
LANES = 16
NC, NS = 2, 16
NW = NC * NS
CHUNK = 64
HC = CHUNK // 2


def _emb_kernel(B):
    bpw = B // NW
    nchunk = bpw // CHUNK
    assert nchunk >= 4 and nchunk % 2 == 0
    mesh = plsc.VectorSubcoreMesh(core_axis_name="c", subcore_axis_name="s")

    @functools.partial(
        pl.kernel,
        mesh=mesh,
        out_type=jax.ShapeDtypeStruct((B, D), jnp.float32),
        scratch_types=[
            pltpu.VMEM((bpw,), jnp.int32),
            pltpu.VMEM((2, CHUNK, D), jnp.float32),
            pltpu.SemaphoreType.DMA,
            pltpu.SemaphoreType.DMA,
            pltpu.SemaphoreType.DMA,
            pltpu.SemaphoreType.DMA,
        ],
    )
    def k(idx_hbm, table_hbm, out_hbm, idx_v, rows, g0, g1, s0, s1):
        gsem = (g0, g1)
        ssem = (s0, s1)
        wid = lax.axis_index("s") * NC + lax.axis_index("c")
        base = wid * bpw
        pltpu.sync_copy(idx_hbm.at[pl.ds(base, bpw)], idx_v)

        def gather(i, b):
            return pltpu.make_async_copy(
                table_hbm.at[idx_v.at[pl.ds(i * CHUNK, CHUNK)]],
                rows.at[b], gsem[b],
            )

        def gather_half(i, b, h):
            return pltpu.make_async_copy(
                table_hbm.at[idx_v.at[pl.ds(i * CHUNK + h * HC, HC)]],
                rows.at[b, pl.ds(h * HC, HC)], gsem[b],
            )

        def scatter_half(i, b, h):
            return pltpu.make_async_copy(
                rows.at[b, pl.ds(h * HC, HC)],
                out_hbm.at[pl.ds(base + i * CHUNK + h * HC, HC)], ssem[b],
            )

        def scale_half(b, h):
            def row_body(r, c):
                for j in range(D // LANES):
                    sl = pl.ds(j * LANES, LANES)
                    rows[b, r, sl] = rows[b, r, sl] * SCALE
                return c

            lax.fori_loop(h * HC, (h + 1) * HC, row_body, 0)

        def process(i, b):
            for h in (0, 1):
                scale_half(b, h)
                scatter_half(i, b, h).start()

        def wait_chunk_scatter(i, b):
            for h in (0, 1):
                scatter_half(i, b, h).wait()

        gather_half(0, 0, 0).start()
        gather_half(0, 0, 1).start()
        gather(1, 1).start()
        for h in (0, 1):
            gather_half(0, 0, h).wait()
            scale_half(0, h)
            scatter_half(0, 0, h).start()

        def pair_body(t, carry):
            for b in (1, 0):
                i = 2 * t + 1 + (1 - b)
                gather(i, b).wait()
                wait_chunk_scatter(i - 1, 1 - b)
                gather(i + 1, 1 - b).start()
                process(i, b)
            return carry

        lax.fori_loop(0, nchunk // 2 - 1, pair_body, 0)

        gather(nchunk - 1, 1).wait()
        wait_chunk_scatter(nchunk - 2, 0)
        process(nchunk - 1, 1)
        wait_chunk_scatter(nchunk - 1, 1)

    return k


def kernel(x, weight):
    b, s = x.shape
    idx = x.reshape(-1).astype(jnp.int32)
    out = _emb_kernel(b * s)(idx, weight)
    return out.reshape(b, s, D)
